# baseline (speedup 1.0000x reference)
import functools

import jax
import jax.numpy as jnp
from jax import lax
from jax.experimental import pallas as pl
from jax.experimental.pallas import tpu as pltpu

NEXP = 8
HID = 128


def _node_moe_body(DB_use,
                   nf_ref, msA_ref, msB_ref, deg_ref,
                   Wg1, Wg2, Wg3, bg,
                   We1, We2, We3, be,
                   WgS, WgD, bgS,
                   WeP, WeQ, beE,
                   hn_out, P_out, Q_out, Gs_out, Gd_out):
  nf = nf_ref[...]
  invd = 1.0 / jnp.maximum(deg_ref[...], 1.0)
  hA = (msA_ref[0] + msA_ref[1]) * invd
  hB = (msB_ref[0, :, :DB_use] + msB_ref[1, :, :DB_use]) * invd
  logits = nf @ Wg1[...] + hA @ Wg2[...] + hB @ Wg3[...] + bg[...]
  blk = logits.shape[0]
  ids = lax.broadcasted_iota(jnp.int32, (blk, NEXP), 1)
  v1 = jnp.max(logits, axis=-1, keepdims=True)
  i1 = jnp.argmax(logits, axis=-1).astype(jnp.int32)[:, None]
  l2 = jnp.where(ids == i1, -jnp.inf, logits)
  v2 = jnp.max(l2, axis=-1, keepdims=True)
  i2 = jnp.argmax(l2, axis=-1).astype(jnp.int32)[:, None]
  e2 = jnp.exp(v2 - v1)
  den = 1.0 + e2
  g1 = 1.0 / den
  g2 = e2 / den
  acc = jnp.zeros((blk, HID), jnp.float32)
  for e in range(NEXP):
    xe = nf @ We1[e] + hA @ We2[e] + hB @ We3[e] + be[0, e]
    w = jnp.where(i1 == e, g1, 0.0) + jnp.where(i2 == e, g2, 0.0)
    acc = acc + w * xe
  hn = jnp.maximum(acc, 0.0)
  hn_out[...] = hn
  for e in range(NEXP):
    P_out[e] = hn @ WeP[e] + 0.5 * beE[0, e]
    Q_out[e] = hn @ WeQ[e] + 0.5 * beE[0, e]
  Gs_out[...] = hn @ WgS[...] + bgS[...]
  Gd_out[...] = hn @ WgD[...]


def _node_moe_prec(nf, msA, msB, deg, Wg_a, bg_a, We_a, be_a,
                   Wg_e, bg_e, We_e, be_e, DB_use):
  N = nf.shape[0]
  DB = msB.shape[2]
  DM = Wg_a.shape[0]
  BLK = 400 if N % 400 == 0 else N
  grid = (N // BLK,)
  Wg1 = Wg_a[:128]
  Wg2 = Wg_a[128:256]
  Wg3 = Wg_a[256:DM]
  We1 = We_a[:, :128, :]
  We2 = We_a[:, 128:256, :]
  We3 = We_a[:, 256:DM, :]
  WgS = jnp.pad(Wg_e[:128], ((0, 0), (0, 8)))
  WgD = jnp.pad(Wg_e[128:], ((0, 0), (0, 8)))
  bgS = jnp.pad(bg_e, (0, 8))[None]
  WeP = We_e[:, :128, :]
  WeQ = We_e[:, 128:, :]

  full = lambda a: pl.BlockSpec(a.shape, lambda i: (0,) * a.ndim)
  outs = pl.pallas_call(
      functools.partial(_node_moe_body, DB_use),
      grid=grid,
      in_specs=[
          pl.BlockSpec((BLK, 128), lambda i: (i, 0)),
          pl.BlockSpec((2, BLK, 128), lambda i: (0, i, 0)),
          pl.BlockSpec((2, BLK, DB), lambda i: (0, i, 0)),
          pl.BlockSpec((BLK, 1), lambda i: (i, 0)),
          full(Wg1), full(Wg2), full(Wg3), full(bg_a[None]),
          full(We1), full(We2), full(We3), full(be_a[None]),
          full(WgS), full(WgD), full(bgS),
          full(WeP), full(WeQ), full(be_e[None]),
      ],
      out_specs=[
          pl.BlockSpec((BLK, 128), lambda i: (i, 0)),
          pl.BlockSpec((NEXP, BLK, 128), lambda i: (0, i, 0)),
          pl.BlockSpec((NEXP, BLK, 128), lambda i: (0, i, 0)),
          pl.BlockSpec((BLK, 16), lambda i: (i, 0)),
          pl.BlockSpec((BLK, 16), lambda i: (i, 0)),
      ],
      out_shape=[
          jax.ShapeDtypeStruct((N, 128), jnp.float32),
          jax.ShapeDtypeStruct((NEXP, N, 128), jnp.float32),
          jax.ShapeDtypeStruct((NEXP, N, 128), jnp.float32),
          jax.ShapeDtypeStruct((N, 16), jnp.float32),
          jax.ShapeDtypeStruct((N, 16), jnp.float32),
      ],
  )(nf, msA, msB, deg, Wg1, Wg2, Wg3, bg_a[None], We1, We2, We3, be_a[None],
    WgS, WgD, bgS, WeP, WeQ, be_e[None])
  return outs


def _seg_gather_scatter(table, src, dst, n):
  s = jax.ops.segment_sum(table[src], dst, num_segments=n)
  return jnp.stack([s, jnp.zeros_like(s)])


def _seg_scatter(rows, dst, n):
  s = jax.ops.segment_sum(rows, dst, num_segments=n)
  return jnp.stack([s, jnp.zeros_like(s)])


def _edge_combine(src, dst, Gs, Gd, P, Q, n):
  logits = Gs[src, :NEXP] + Gd[dst, :NEXP]
  v, i = lax.top_k(logits, 2)
  g = jax.nn.softmax(v, axis=-1)
  Pf = P.reshape(NEXP * n, 128)
  Qf = Q.reshape(NEXP * n, 128)
  r1 = Pf[i[:, 0] * n + src] + Qf[i[:, 0] * n + dst]
  r2 = Pf[i[:, 1] * n + src] + Qf[i[:, 1] * n + dst]
  return jnp.maximum(g[:, :1] * r1 + g[:, 1:] * r2, 0.0)


def kernel(edge_index, nfeats, efeats,
           Wg_a0, bg_a0, We_a0, be_a0, Wg_e0, bg_e0, We_e0, be_e0,
           Wg_a1, bg_a1, We_a1, be_a1, Wg_e1, bg_e1, We_e1, be_e1):
  n = nfeats.shape[0]
  e = efeats.shape[0]
  src = edge_index[0].astype(jnp.int32)
  dst = edge_index[1].astype(jnp.int32)

  msA0 = _seg_gather_scatter(nfeats, src, dst, n)
  ef_aug = jnp.concatenate(
      [efeats, jnp.ones((e, 1), jnp.float32),
       jnp.zeros((e, 15), jnp.float32)], axis=1)
  msB0 = _seg_scatter(ef_aug, dst, n)
  deg = (msB0[0, :, 16] + msB0[1, :, 16])[:, None]

  hn0, P0, Q0, Gs0, Gd0 = _node_moe_prec(
      nfeats, msA0, msB0, deg, Wg_a0, bg_a0, We_a0, be_a0,
      Wg_e0, bg_e0, We_e0, be_e0, DB_use=16)
  he0 = _edge_combine(src, dst, Gs0, Gd0, P0, Q0, n)

  msA1 = _seg_gather_scatter(hn0, src, dst, n)
  msB1 = _seg_scatter(he0, dst, n)
  hn1, P1, Q1, Gs1, Gd1 = _node_moe_prec(
      hn0, msA1, msB1, deg, Wg_a1, bg_a1, We_a1, be_a1,
      Wg_e1, bg_e1, We_e1, be_e1, DB_use=128)
  he1 = _edge_combine(src, dst, Gs1, Gd1, P1, Q1, n)
  return (hn1, he1)


# speedup vs baseline: 35.6474x; 35.6474x over previous
import functools

import jax
import jax.numpy as jnp
from jax import lax
from jax.experimental import pallas as pl
from jax.experimental.pallas import tpu as pltpu
from jax.experimental.pallas import tpu_sc as plsc

NEXP = 8
HID = 128
LANES = 16


def _sc_mesh():
  return plsc.VectorSubcoreMesh(core_axis_name="c", subcore_axis_name="s")


def _sc_info():
  try:
    info = plsc.get_sparse_core_info()
    return info.num_cores, info.num_subcores
  except Exception:
    return 2, 16


def _node_moe_body(DB_use,
                   nf_ref, msA_ref, msB_ref, deg_ref,
                   Wg1, Wg2, Wg3, bg,
                   We1, We2, We3, be,
                   WgS, WgD, bgS,
                   WeP, WeQ, beE,
                   hn_out, P_out, Q_out):
  nf = nf_ref[...]
  invd = 1.0 / jnp.maximum(deg_ref[...], 1.0)
  hA = (msA_ref[0] + msA_ref[1]) * invd
  hB = (msB_ref[0, :, :DB_use] + msB_ref[1, :, :DB_use]) * invd
  logits = nf @ Wg1[...] + hA @ Wg2[...] + hB @ Wg3[...] + bg[...]
  blk = logits.shape[0]
  ids = lax.broadcasted_iota(jnp.int32, (blk, NEXP), 1)
  v1 = jnp.max(logits, axis=-1, keepdims=True)
  i1 = jnp.argmax(logits, axis=-1).astype(jnp.int32)[:, None]
  l2 = jnp.where(ids == i1, -jnp.inf, logits)
  v2 = jnp.max(l2, axis=-1, keepdims=True)
  i2 = jnp.argmax(l2, axis=-1).astype(jnp.int32)[:, None]
  e2 = jnp.exp(v2 - v1)
  den = 1.0 + e2
  g1 = 1.0 / den
  g2 = e2 / den
  acc = jnp.zeros((blk, HID), jnp.float32)
  for e in range(NEXP):
    xe = nf @ We1[e] + hA @ We2[e] + hB @ We3[e] + be[0, e]
    w = jnp.where(i1 == e, g1, 0.0) + jnp.where(i2 == e, g2, 0.0)
    acc = acc + w * xe
  hn = jnp.maximum(acc, 0.0)
  hn_out[...] = hn
  for e in range(NEXP):
    P_out[e] = hn @ WeP[e] + 0.5 * beE[0, e]
    Q_out[e] = hn @ WeQ[e] + 0.5 * beE[0, e]
  P_out[NEXP] = hn @ WgS[...] + bgS[...]
  Q_out[NEXP] = hn @ WgD[...]


def _node_moe_prec(nf, msA, msB, deg, Wg_a, bg_a, We_a, be_a,
                   Wg_e, bg_e, We_e, be_e, DB_use):
  N = nf.shape[0]
  DB = msB.shape[2]
  DM = Wg_a.shape[0]
  BLK = 400 if N % 400 == 0 else N
  grid = (N // BLK,)
  Wg1 = Wg_a[:128]
  Wg2 = Wg_a[128:256]
  Wg3 = Wg_a[256:DM]
  We1 = We_a[:, :128, :]
  We2 = We_a[:, 128:256, :]
  We3 = We_a[:, 256:DM, :]
  WgS = jnp.pad(Wg_e[:128], ((0, 0), (0, 120)))
  WgD = jnp.pad(Wg_e[128:], ((0, 0), (0, 120)))
  bgS = jnp.pad(bg_e, (0, 120))[None]
  WeP = We_e[:, :128, :]
  WeQ = We_e[:, 128:, :]

  full = lambda a: pl.BlockSpec(a.shape, lambda i: (0,) * a.ndim)
  outs = pl.pallas_call(
      functools.partial(_node_moe_body, DB_use),
      grid=grid,
      in_specs=[
          pl.BlockSpec((BLK, 128), lambda i: (i, 0)),
          pl.BlockSpec((2, BLK, 128), lambda i: (0, i, 0)),
          pl.BlockSpec((2, BLK, DB), lambda i: (0, i, 0)),
          pl.BlockSpec((BLK, 1), lambda i: (i, 0)),
          full(Wg1), full(Wg2), full(Wg3), full(bg_a[None]),
          full(We1), full(We2), full(We3), full(be_a[None]),
          full(WgS), full(WgD), full(bgS),
          full(WeP), full(WeQ), full(be_e[None]),
      ],
      out_specs=[
          pl.BlockSpec((BLK, 128), lambda i: (i, 0)),
          pl.BlockSpec((NEXP + 1, BLK, 128), lambda i: (0, i, 0)),
          pl.BlockSpec((NEXP + 1, BLK, 128), lambda i: (0, i, 0)),
      ],
      out_shape=[
          jax.ShapeDtypeStruct((N, 128), jnp.float32),
          jax.ShapeDtypeStruct((NEXP + 1, N, 128), jnp.float32),
          jax.ShapeDtypeStruct((NEXP + 1, N, 128), jnp.float32),
      ],
  )(nf, msA, msB, deg, Wg1, Wg2, Wg3, bg_a[None], We1, We2, We3, be_a[None],
    WgS, WgD, bgS, WeP, WeQ, be_e[None])
  return outs


def _seg_gather_scatter(table, src, dst, n):
  s = jax.ops.segment_sum(table[src], dst, num_segments=n)
  return jnp.stack([s, jnp.zeros_like(s)])


def _seg_scatter(rows, dst, n):
  s = jax.ops.segment_sum(rows, dst, num_segments=n)
  return jnp.stack([s, jnp.zeros_like(s)])


def _edge_combine(src, dst, P, Q, n):
  E = src.shape[0]
  NC, NS = _sc_info()
  NW = NC * NS
  EW = E // NW
  C = 80
  NCH = EW // C
  Pf = P.reshape((NEXP + 1) * n, HID)
  Qf = Q.reshape((NEXP + 1) * n, HID)

  @functools.partial(
      pl.kernel,
      out_type=jax.ShapeDtypeStruct((E * HID,), jnp.float32),
      mesh=_sc_mesh(),
      scratch_types=[
          pltpu.VMEM((C,), jnp.int32),
          pltpu.VMEM((C,), jnp.int32),
          pltpu.VMEM((C, HID), jnp.float32),
          pltpu.VMEM((C, HID), jnp.float32),
          pltpu.VMEM((C,), jnp.float32),
          pltpu.VMEM((C,), jnp.float32),
          pltpu.VMEM((C,), jnp.int32),
          pltpu.VMEM((C,), jnp.int32),
          pltpu.VMEM((C,), jnp.int32),
          pltpu.VMEM((C,), jnp.int32),
          pltpu.VMEM((C, HID), jnp.float32),
          pltpu.VMEM((C, HID), jnp.float32),
          pltpu.VMEM((C, HID), jnp.float32),
          pltpu.VMEM((C, HID), jnp.float32),
          pltpu.VMEM((C * HID,), jnp.float32),
          pltpu.SemaphoreType.DMA,
      ],
  )
  def combine(src_h, dst_h, pf_h, qf_h, out_h,
              src_v, dst_v, gs_v, gd_v, g1_v, g2_v,
              f1_v, f2_v, f3_v, f4_v, b1_v, b2_v, b3_v, b4_v, out_v, sem):
    wid = lax.axis_index("s") * NC + lax.axis_index("c")
    ninf = jnp.full((LANES,), -jnp.inf, jnp.float32)
    zero_i = jnp.zeros((LANES,), jnp.int32)
    iota = lax.iota(jnp.int32, LANES)
    perms = {s: iota ^ s for s in (1, 2, 4, 8)}
    masks = {s: (iota & s) != 0 for s in (1, 2, 4, 8)}

    def _take(v, idx):
      return lax.gather(
          v, idx[:, None],
          lax.GatherDimensionNumbers(offset_dims=(), collapsed_slice_dims=(0,),
                                     start_index_map=(0,)),
          slice_sizes=(1,),
          mode=lax.GatherScatterMode.PROMISE_IN_BOUNDS)

    def chunk_body(ci, carry):
      base = wid * EW + ci * C
      pltpu.sync_copy(src_h.at[pl.ds(base, C)], src_v)
      pltpu.sync_copy(dst_h.at[pl.ds(base, C)], dst_v)
      for g in range(C // LANES):
        sl = pl.ds(g * LANES, LANES)
        f1_v[sl] = src_v[sl] + NEXP * n
        f2_v[sl] = dst_v[sl] + NEXP * n
      pltpu.async_copy(pf_h.at[f1_v], gs_v, sem).wait()
      pltpu.async_copy(qf_h.at[f2_v], gd_v, sem).wait()
      for g in range(C // LANES):
        R = [gs_v[g * LANES + l, pl.ds(0, LANES)]
             + gd_v[g * LANES + l, pl.ds(0, LANES)]
             for l in range(LANES)]
        for s in (8, 4, 2, 1):
          pm, mk = perms[s], masks[s]
          for i in range(LANES):
            if i & s:
              continue
            a, b = R[i], R[i | s]
            R[i] = jnp.where(mk, _take(b, pm), a)
            R[i | s] = jnp.where(mk, b, _take(a, pm))
        m1 = R[0]
        i1 = zero_i
        m2 = ninf
        i2 = zero_i
        for j in range(1, NEXP):
          x = R[j]
          cj = zero_i + j
          gt1 = x > m1
          gt2 = x > m2
          i2 = jnp.where(gt1, i1, jnp.where(gt2, cj, i2))
          m2 = jnp.where(gt1, m1, jnp.where(gt2, x, m2))
          i1 = jnp.where(gt1, cj, i1)
          m1 = jnp.where(gt1, x, m1)
        e2 = jnp.exp(m2 - m1)
        g1 = 1.0 / (1.0 + e2)
        sl = pl.ds(g * LANES, LANES)
        s16 = src_v[sl]
        d16 = dst_v[sl]
        g1_v[sl] = g1
        g2_v[sl] = 1.0 - g1
        f1_v[sl] = i1 * n + s16
        f2_v[sl] = i1 * n + d16
        f3_v[sl] = i2 * n + s16
        f4_v[sl] = i2 * n + d16
      pltpu.async_copy(pf_h.at[f1_v], b1_v, sem).wait()
      pltpu.async_copy(qf_h.at[f2_v], b2_v, sem).wait()
      pltpu.async_copy(pf_h.at[f3_v], b3_v, sem).wait()
      pltpu.async_copy(qf_h.at[f4_v], b4_v, sem).wait()

      def edge_body(e, c2):
        b16 = (e // LANES) * LANES
        off = e - b16
        offv = zero_i + off
        g1b = _take(g1_v[pl.ds(b16, LANES)], offv)
        g2b = _take(g2_v[pl.ds(b16, LANES)], offv)
        for k in range(HID // LANES):
          sk = pl.ds(k * LANES, LANES)
          r = (g1b * (b1_v[e, sk] + b2_v[e, sk])
               + g2b * (b3_v[e, sk] + b4_v[e, sk]))
          out_v[pl.ds(e * HID + k * LANES, LANES)] = jnp.maximum(r, 0.0)
        return c2
      lax.fori_loop(0, C, edge_body, 0)
      pltpu.sync_copy(out_v, out_h.at[pl.ds(base * HID, C * HID)])
      return carry

    lax.fori_loop(0, NCH, chunk_body, 0)

  return combine(src, dst, Pf, Qf).reshape(E, HID)


def kernel(edge_index, nfeats, efeats,
           Wg_a0, bg_a0, We_a0, be_a0, Wg_e0, bg_e0, We_e0, be_e0,
           Wg_a1, bg_a1, We_a1, be_a1, Wg_e1, bg_e1, We_e1, be_e1):
  n = nfeats.shape[0]
  e = efeats.shape[0]
  src = edge_index[0].astype(jnp.int32)
  dst = edge_index[1].astype(jnp.int32)

  msA0 = _seg_gather_scatter(nfeats, src, dst, n)
  ef_aug = jnp.concatenate(
      [efeats, jnp.ones((e, 1), jnp.float32),
       jnp.zeros((e, 15), jnp.float32)], axis=1)
  msB0 = _seg_scatter(ef_aug, dst, n)
  deg = (msB0[0, :, 16] + msB0[1, :, 16])[:, None]

  hn0, P0, Q0 = _node_moe_prec(
      nfeats, msA0, msB0, deg, Wg_a0, bg_a0, We_a0, be_a0,
      Wg_e0, bg_e0, We_e0, be_e0, DB_use=16)
  he0 = _edge_combine(src, dst, P0, Q0, n)

  msA1 = _seg_gather_scatter(hn0, src, dst, n)
  msB1 = _seg_scatter(he0, dst, n)
  hn1, P1, Q1 = _node_moe_prec(
      hn0, msA1, msB1, deg, Wg_a1, bg_a1, We_a1, be_a1,
      Wg_e1, bg_e1, We_e1, be_e1, DB_use=128)
  he1 = _edge_combine(src, dst, P1, Q1, n)
  return (hn1, he1)


# speedup vs baseline: 40.3856x; 1.1329x over previous
import functools

import jax
import jax.numpy as jnp
from jax import lax
from jax.experimental import pallas as pl
from jax.experimental.pallas import tpu as pltpu
from jax.experimental.pallas import tpu_sc as plsc

NEXP = 8
HID = 128
LANES = 16


def _sc_mesh():
  return plsc.VectorSubcoreMesh(core_axis_name="c", subcore_axis_name="s")


def _sc_info():
  try:
    info = plsc.get_sparse_core_info()
    return info.num_cores, info.num_subcores
  except Exception:
    return 2, 16


def _node_moe_body(DB_use,
                   nf_ref, msA_ref, msB_ref, deg_ref,
                   Wg1, Wg2, Wg3, bg,
                   We1, We2, We3, be,
                   WgS, WgD, bgS,
                   WeP, WeQ, beE,
                   hn_out, P_out, Q_out):
  nf = nf_ref[...]
  invd = 1.0 / jnp.maximum(deg_ref[...], 1.0)
  hA = sum(msA_ref[k] for k in range(msA_ref.shape[0])) * invd
  hB = sum(msB_ref[k, :, :DB_use] for k in range(msB_ref.shape[0])) * invd
  logits = nf @ Wg1[...] + hA @ Wg2[...] + hB @ Wg3[...] + bg[...]
  blk = logits.shape[0]
  ids = lax.broadcasted_iota(jnp.int32, (blk, NEXP), 1)
  v1 = jnp.max(logits, axis=-1, keepdims=True)
  i1 = jnp.argmax(logits, axis=-1).astype(jnp.int32)[:, None]
  l2 = jnp.where(ids == i1, -jnp.inf, logits)
  v2 = jnp.max(l2, axis=-1, keepdims=True)
  i2 = jnp.argmax(l2, axis=-1).astype(jnp.int32)[:, None]
  e2 = jnp.exp(v2 - v1)
  den = 1.0 + e2
  g1 = 1.0 / den
  g2 = e2 / den
  acc = jnp.zeros((blk, HID), jnp.float32)
  for e in range(NEXP):
    xe = nf @ We1[e] + hA @ We2[e] + hB @ We3[e] + be[0, e]
    w = jnp.where(i1 == e, g1, 0.0) + jnp.where(i2 == e, g2, 0.0)
    acc = acc + w * xe
  hn = jnp.maximum(acc, 0.0)
  hn_out[...] = hn
  for e in range(NEXP):
    P_out[e] = hn @ WeP[e] + 0.5 * beE[0, e]
    Q_out[e] = hn @ WeQ[e] + 0.5 * beE[0, e]
  P_out[NEXP] = hn @ WgS[...] + bgS[...]
  Q_out[NEXP] = hn @ WgD[...]


def _node_moe_prec(nf, msA, msB, deg, Wg_a, bg_a, We_a, be_a,
                   Wg_e, bg_e, We_e, be_e, DB_use):
  N = nf.shape[0]
  DB = msB.shape[2]
  DM = Wg_a.shape[0]
  BLK = 400 if N % 400 == 0 else N
  grid = (N // BLK,)
  Wg1 = Wg_a[:128]
  Wg2 = Wg_a[128:256]
  Wg3 = Wg_a[256:DM]
  We1 = We_a[:, :128, :]
  We2 = We_a[:, 128:256, :]
  We3 = We_a[:, 256:DM, :]
  WgS = jnp.pad(Wg_e[:128], ((0, 0), (0, 120)))
  WgD = jnp.pad(Wg_e[128:], ((0, 0), (0, 120)))
  bgS = jnp.pad(bg_e, (0, 120))[None]
  WeP = We_e[:, :128, :]
  WeQ = We_e[:, 128:, :]

  full = lambda a: pl.BlockSpec(a.shape, lambda i: (0,) * a.ndim)
  outs = pl.pallas_call(
      functools.partial(_node_moe_body, DB_use),
      grid=grid,
      in_specs=[
          pl.BlockSpec((BLK, 128), lambda i: (i, 0)),
          pl.BlockSpec((msA.shape[0], BLK, 128), lambda i: (0, i, 0)),
          pl.BlockSpec((msB.shape[0], BLK, DB), lambda i: (0, i, 0)),
          pl.BlockSpec((BLK, 1), lambda i: (i, 0)),
          full(Wg1), full(Wg2), full(Wg3), full(bg_a[None]),
          full(We1), full(We2), full(We3), full(be_a[None]),
          full(WgS), full(WgD), full(bgS),
          full(WeP), full(WeQ), full(be_e[None]),
      ],
      out_specs=[
          pl.BlockSpec((BLK, 128), lambda i: (i, 0)),
          pl.BlockSpec((NEXP + 1, BLK, 128), lambda i: (0, i, 0)),
          pl.BlockSpec((NEXP + 1, BLK, 128), lambda i: (0, i, 0)),
      ],
      out_shape=[
          jax.ShapeDtypeStruct((N, 128), jnp.float32),
          jax.ShapeDtypeStruct((NEXP + 1, N, 128), jnp.float32),
          jax.ShapeDtypeStruct((NEXP + 1, N, 128), jnp.float32),
      ],
  )(nf, msA, msB, deg, Wg1, Wg2, Wg3, bg_a[None], We1, We2, We3, be_a[None],
    WgS, WgD, bgS, WeP, WeQ, be_e[None])
  return outs


def _seg_gather_scatter_jnp(table, src, dst, n):
  s = jax.ops.segment_sum(table[src], dst, num_segments=n)
  return s[None]


def _seg_scatter_jnp(rows, dst, n):
  s = jax.ops.segment_sum(rows, dst, num_segments=n)
  return s[None]


def _seg_gather_scatter(table, src, dst, n):
  E = src.shape[0]
  NC, NS = _sc_info()
  HALF = n // NC
  ACC = HALF + 16
  ET = E // NS
  C2 = 80
  ZR = (ACC // NS) // 8 * 8
  ZTAIL = ACC - NS * ZR
  OR_ = (HALF // NS) // 8 * 8
  OTAIL = HALF - NS * OR_
  zeros = jnp.zeros((max(ZR, ZTAIL), 128), jnp.float32)

  @functools.partial(
      pl.kernel,
      out_type=jax.ShapeDtypeStruct((n, 128), jnp.float32),
      mesh=_sc_mesh(),
      scratch_types=[
          pltpu.VMEM((C2,), jnp.int32),
          pltpu.VMEM((C2,), jnp.int32),
          pltpu.VMEM((C2,), jnp.int32),
          pltpu.VMEM((C2, 128), jnp.float32),
          pltpu.VMEM_SHARED((ACC, 128), jnp.float32),
          pltpu.SemaphoreType.DMA,
      ],
  )
  def seg(table_h, src_h, dst_h, z_h, out_h, src_v, dst_v, dloc_v, rows_v,
          acc_sh, sem):
    c = lax.axis_index("c")
    sid = lax.axis_index("s")
    lo = c * HALF
    pltpu.sync_copy(z_h.at[pl.ds(0, ZR)], acc_sh.at[pl.ds(sid * ZR, ZR)])

    @pl.when(sid == NS - 1)
    def _zero_tail():
      pltpu.sync_copy(z_h.at[pl.ds(0, ZTAIL)],
                      acc_sh.at[pl.ds(NS * ZR, ZTAIL)])

    plsc.subcore_barrier()

    def chunk(ci, carry):
      base = sid * ET + ci * C2
      pltpu.sync_copy(src_h.at[pl.ds(base, C2)], src_v)
      pltpu.sync_copy(dst_h.at[pl.ds(base, C2)], dst_v)
      for g in range(C2 // LANES):
        sl = pl.ds(g * LANES, LANES)
        loc = dst_v[sl] - lo
        ok = (loc >= 0) & (loc < HALF)
        dloc_v[sl] = jnp.where(ok, loc, HALF)
      pltpu.async_copy(table_h.at[src_v], rows_v, sem).wait()
      pltpu.sync_copy(rows_v, acc_sh.at[dloc_v], add=True)
      return carry

    lax.fori_loop(0, ET // C2, chunk, 0)
    plsc.subcore_barrier()
    pltpu.sync_copy(acc_sh.at[pl.ds(sid * OR_, OR_)],
                    out_h.at[pl.ds(lo + sid * OR_, OR_)])

    @pl.when(sid == NS - 1)
    def _out_tail():
      pltpu.sync_copy(acc_sh.at[pl.ds(NS * OR_, OTAIL)],
                      out_h.at[pl.ds(lo + NS * OR_, OTAIL)])

  return seg(table, src, dst, zeros)[None]


def _seg_scatter(rows, dst, n):
  E, D = rows.shape
  NC, NS = _sc_info()
  HALF = n // NC
  ACC = HALF + 16
  ET = E // NS
  C2 = 80
  ZR = (ACC // NS) // 8 * 8
  ZTAIL = ACC - NS * ZR
  OR_ = (HALF // NS) // 8 * 8
  OTAIL = HALF - NS * OR_
  zeros = jnp.zeros((max(ZR, ZTAIL, C2), 128), jnp.float32)

  @functools.partial(
      pl.kernel,
      out_type=jax.ShapeDtypeStruct((n, 128), jnp.float32),
      mesh=_sc_mesh(),
      scratch_types=[
          pltpu.VMEM((C2,), jnp.int32),
          pltpu.VMEM((C2,), jnp.int32),
          pltpu.VMEM((C2, D), jnp.float32),
          pltpu.VMEM((C2, 128), jnp.float32),
          pltpu.VMEM_SHARED((ACC, 128), jnp.float32),
          pltpu.SemaphoreType.DMA,
      ],
  )
  def seg(rows_h, dst_h, z_h, out_h, dst_v, dloc_v, rows_s, rows_v, acc_sh,
          sem):
    c = lax.axis_index("c")
    sid = lax.axis_index("s")
    lo = c * HALF
    pltpu.sync_copy(z_h.at[pl.ds(0, ZR)], acc_sh.at[pl.ds(sid * ZR, ZR)])

    @pl.when(sid == NS - 1)
    def _zero_tail():
      pltpu.sync_copy(z_h.at[pl.ds(0, ZTAIL)],
                      acc_sh.at[pl.ds(NS * ZR, ZTAIL)])

    if D < 128:
      pltpu.sync_copy(z_h.at[pl.ds(0, C2)], rows_v)
    plsc.subcore_barrier()

    def chunk(ci, carry):
      base = sid * ET + ci * C2
      pltpu.sync_copy(dst_h.at[pl.ds(base, C2)], dst_v)
      for g in range(C2 // LANES):
        sl = pl.ds(g * LANES, LANES)
        loc = dst_v[sl] - lo
        ok = (loc >= 0) & (loc < HALF)
        dloc_v[sl] = jnp.where(ok, loc, HALF)
      if D < 128:
        pltpu.sync_copy(rows_h.at[pl.ds(base, C2)], rows_s)

        def expand(e, c2):
          for k in range(D // LANES):
            rows_v[e, pl.ds(k * LANES, LANES)] = rows_s[e,
                                                        pl.ds(k * LANES,
                                                              LANES)]
          return c2

        lax.fori_loop(0, C2, expand, 0)
      else:
        pltpu.sync_copy(rows_h.at[pl.ds(base, C2)], rows_v)
      pltpu.sync_copy(rows_v, acc_sh.at[dloc_v], add=True)
      return carry

    lax.fori_loop(0, ET // C2, chunk, 0)
    plsc.subcore_barrier()
    pltpu.sync_copy(acc_sh.at[pl.ds(sid * OR_, OR_)],
                    out_h.at[pl.ds(lo + sid * OR_, OR_)])

    @pl.when(sid == NS - 1)
    def _out_tail():
      pltpu.sync_copy(acc_sh.at[pl.ds(NS * OR_, OTAIL)],
                      out_h.at[pl.ds(lo + NS * OR_, OTAIL)])

  return seg(rows, dst, zeros)[None]


def _edge_combine(src, dst, P, Q, n):
  E = src.shape[0]
  NC, NS = _sc_info()
  NW = NC * NS
  EW = E // NW
  C = 80
  NCH = EW // C
  Pf = P.reshape((NEXP + 1) * n, HID)
  Qf = Q.reshape((NEXP + 1) * n, HID)

  @functools.partial(
      pl.kernel,
      out_type=jax.ShapeDtypeStruct((E * HID,), jnp.float32),
      mesh=_sc_mesh(),
      scratch_types=[
          pltpu.VMEM((C,), jnp.int32),
          pltpu.VMEM((C,), jnp.int32),
          pltpu.VMEM((C, HID), jnp.float32),
          pltpu.VMEM((C, HID), jnp.float32),
          pltpu.VMEM((C,), jnp.float32),
          pltpu.VMEM((C,), jnp.float32),
          pltpu.VMEM((C,), jnp.int32),
          pltpu.VMEM((C,), jnp.int32),
          pltpu.VMEM((C,), jnp.int32),
          pltpu.VMEM((C,), jnp.int32),
          pltpu.VMEM((C, HID), jnp.float32),
          pltpu.VMEM((C, HID), jnp.float32),
          pltpu.VMEM((C, HID), jnp.float32),
          pltpu.VMEM((C, HID), jnp.float32),
          pltpu.VMEM((C * HID,), jnp.float32),
          pltpu.SemaphoreType.DMA,
      ],
  )
  def combine(src_h, dst_h, pf_h, qf_h, out_h,
              src_v, dst_v, gs_v, gd_v, g1_v, g2_v,
              f1_v, f2_v, f3_v, f4_v, b1_v, b2_v, b3_v, b4_v, out_v, sem):
    wid = lax.axis_index("s") * NC + lax.axis_index("c")
    ninf = jnp.full((LANES,), -jnp.inf, jnp.float32)
    zero_i = jnp.zeros((LANES,), jnp.int32)
    iota = lax.iota(jnp.int32, LANES)
    perms = {s: iota ^ s for s in (1, 2, 4, 8)}
    masks = {s: (iota & s) != 0 for s in (1, 2, 4, 8)}

    def _take(v, idx):
      return lax.gather(
          v, idx[:, None],
          lax.GatherDimensionNumbers(offset_dims=(), collapsed_slice_dims=(0,),
                                     start_index_map=(0,)),
          slice_sizes=(1,),
          mode=lax.GatherScatterMode.PROMISE_IN_BOUNDS)

    def chunk_body(ci, carry):
      base = wid * EW + ci * C
      pltpu.sync_copy(src_h.at[pl.ds(base, C)], src_v)
      pltpu.sync_copy(dst_h.at[pl.ds(base, C)], dst_v)
      for g in range(C // LANES):
        sl = pl.ds(g * LANES, LANES)
        f1_v[sl] = src_v[sl] + NEXP * n
        f2_v[sl] = dst_v[sl] + NEXP * n
      pltpu.async_copy(pf_h.at[f1_v], gs_v, sem).wait()
      pltpu.async_copy(qf_h.at[f2_v], gd_v, sem).wait()
      for g in range(C // LANES):
        R = [gs_v[g * LANES + l, pl.ds(0, LANES)]
             + gd_v[g * LANES + l, pl.ds(0, LANES)]
             for l in range(LANES)]
        for s in (8, 4, 2, 1):
          pm, mk = perms[s], masks[s]
          for i in range(LANES):
            if i & s:
              continue
            a, b = R[i], R[i | s]
            R[i] = jnp.where(mk, _take(b, pm), a)
            R[i | s] = jnp.where(mk, b, _take(a, pm))
        m1 = R[0]
        i1 = zero_i
        m2 = ninf
        i2 = zero_i
        for j in range(1, NEXP):
          x = R[j]
          cj = zero_i + j
          gt1 = x > m1
          gt2 = x > m2
          i2 = jnp.where(gt1, i1, jnp.where(gt2, cj, i2))
          m2 = jnp.where(gt1, m1, jnp.where(gt2, x, m2))
          i1 = jnp.where(gt1, cj, i1)
          m1 = jnp.where(gt1, x, m1)
        e2 = jnp.exp(m2 - m1)
        g1 = 1.0 / (1.0 + e2)
        sl = pl.ds(g * LANES, LANES)
        s16 = src_v[sl]
        d16 = dst_v[sl]
        g1_v[sl] = g1
        g2_v[sl] = 1.0 - g1
        f1_v[sl] = i1 * n + s16
        f2_v[sl] = i1 * n + d16
        f3_v[sl] = i2 * n + s16
        f4_v[sl] = i2 * n + d16
      pltpu.async_copy(pf_h.at[f1_v], b1_v, sem).wait()
      pltpu.async_copy(qf_h.at[f2_v], b2_v, sem).wait()
      pltpu.async_copy(pf_h.at[f3_v], b3_v, sem).wait()
      pltpu.async_copy(qf_h.at[f4_v], b4_v, sem).wait()

      def edge_body(e, c2):
        b16 = (e // LANES) * LANES
        off = e - b16
        offv = zero_i + off
        g1b = _take(g1_v[pl.ds(b16, LANES)], offv)
        g2b = _take(g2_v[pl.ds(b16, LANES)], offv)
        for k in range(HID // LANES):
          sk = pl.ds(k * LANES, LANES)
          r = (g1b * (b1_v[e, sk] + b2_v[e, sk])
               + g2b * (b3_v[e, sk] + b4_v[e, sk]))
          out_v[pl.ds(e * HID + k * LANES, LANES)] = jnp.maximum(r, 0.0)
        return c2
      lax.fori_loop(0, C, edge_body, 0)
      pltpu.sync_copy(out_v, out_h.at[pl.ds(base * HID, C * HID)])
      return carry

    lax.fori_loop(0, NCH, chunk_body, 0)

  return combine(src, dst, Pf, Qf).reshape(E, HID)


def kernel(edge_index, nfeats, efeats,
           Wg_a0, bg_a0, We_a0, be_a0, Wg_e0, bg_e0, We_e0, be_e0,
           Wg_a1, bg_a1, We_a1, be_a1, Wg_e1, bg_e1, We_e1, be_e1):
  n = nfeats.shape[0]
  e = efeats.shape[0]
  src = edge_index[0].astype(jnp.int32)
  dst = edge_index[1].astype(jnp.int32)

  msA0 = _seg_gather_scatter(nfeats, src, dst, n)
  ef_aug = jnp.concatenate(
      [efeats, jnp.ones((e, 1), jnp.float32),
       jnp.zeros((e, 15), jnp.float32)], axis=1)
  msB0 = _seg_scatter(ef_aug, dst, n)
  deg = msB0.sum(axis=0)[:, 16:17]

  hn0, P0, Q0 = _node_moe_prec(
      nfeats, msA0, msB0, deg, Wg_a0, bg_a0, We_a0, be_a0,
      Wg_e0, bg_e0, We_e0, be_e0, DB_use=16)
  he0 = _edge_combine(src, dst, P0, Q0, n)

  msA1 = _seg_gather_scatter(hn0, src, dst, n)
  msB1 = _seg_scatter(he0, dst, n)
  hn1, P1, Q1 = _node_moe_prec(
      hn0, msA1, msB1, deg, Wg_a1, bg_a1, We_a1, be_a1,
      Wg_e1, bg_e1, We_e1, be_e1, DB_use=128)
  he1 = _edge_combine(src, dst, P1, Q1, n)
  return (hn1, he1)


# speedup vs baseline: 49.4218x; 1.2237x over previous
import functools

import jax
import jax.numpy as jnp
from jax import lax
from jax.experimental import pallas as pl
from jax.experimental.pallas import tpu as pltpu
from jax.experimental.pallas import tpu_sc as plsc

NEXP = 8
HID = 128
LANES = 16


def _sc_mesh():
  return plsc.VectorSubcoreMesh(core_axis_name="c", subcore_axis_name="s")


def _sc_info():
  try:
    info = plsc.get_sparse_core_info()
    return info.num_cores, info.num_subcores
  except Exception:
    return 2, 16


def _node_moe_body(DB_use,
                   nf_ref, msA_ref, msB_ref, deg_ref,
                   Wg1, Wg2, Wg3, bg,
                   We1, We2, We3, be,
                   WgS, WgD, bgS,
                   WeP, WeQ, beE,
                   hn_out, P_out, Q_out):
  nf = nf_ref[...]
  invd = 1.0 / jnp.maximum(deg_ref[...], 1.0)
  hA = sum(msA_ref[k] for k in range(msA_ref.shape[0])) * invd
  hB = sum(msB_ref[k, :, :DB_use] for k in range(msB_ref.shape[0])) * invd
  logits = nf @ Wg1[...] + hA @ Wg2[...] + hB @ Wg3[...] + bg[...]
  blk = logits.shape[0]
  ids = lax.broadcasted_iota(jnp.int32, (blk, NEXP), 1)
  v1 = jnp.max(logits, axis=-1, keepdims=True)
  i1 = jnp.argmax(logits, axis=-1).astype(jnp.int32)[:, None]
  l2 = jnp.where(ids == i1, -jnp.inf, logits)
  v2 = jnp.max(l2, axis=-1, keepdims=True)
  i2 = jnp.argmax(l2, axis=-1).astype(jnp.int32)[:, None]
  e2 = jnp.exp(v2 - v1)
  den = 1.0 + e2
  g1 = 1.0 / den
  g2 = e2 / den
  acc = jnp.zeros((blk, HID), jnp.float32)
  for e in range(NEXP):
    xe = nf @ We1[e] + hA @ We2[e] + hB @ We3[e] + be[0, e]
    w = jnp.where(i1 == e, g1, 0.0) + jnp.where(i2 == e, g2, 0.0)
    acc = acc + w * xe
  hn = jnp.maximum(acc, 0.0)
  hn_out[...] = hn
  for e in range(NEXP):
    P_out[e] = hn @ WeP[e] + 0.5 * beE[0, e]
    Q_out[e] = hn @ WeQ[e] + 0.5 * beE[0, e]
  P_out[NEXP] = hn @ WgS[...] + bgS[...]
  Q_out[NEXP] = hn @ WgD[...]


def _node_moe_prec(nf, msA, msB, deg, Wg_a, bg_a, We_a, be_a,
                   Wg_e, bg_e, We_e, be_e, DB_use):
  N = nf.shape[0]
  DB = msB.shape[2]
  DM = Wg_a.shape[0]
  BLK = 400 if N % 400 == 0 else N
  grid = (N // BLK,)
  Wg1 = Wg_a[:128]
  Wg2 = Wg_a[128:256]
  Wg3 = Wg_a[256:DM]
  We1 = We_a[:, :128, :]
  We2 = We_a[:, 128:256, :]
  We3 = We_a[:, 256:DM, :]
  WgS = jnp.pad(Wg_e[:128], ((0, 0), (0, 120)))
  WgD = jnp.pad(Wg_e[128:], ((0, 0), (0, 120)))
  bgS = jnp.pad(bg_e, (0, 120))[None]
  WeP = We_e[:, :128, :]
  WeQ = We_e[:, 128:, :]

  full = lambda a: pl.BlockSpec(a.shape, lambda i: (0,) * a.ndim)
  outs = pl.pallas_call(
      functools.partial(_node_moe_body, DB_use),
      grid=grid,
      in_specs=[
          pl.BlockSpec((BLK, 128), lambda i: (i, 0)),
          pl.BlockSpec((msA.shape[0], BLK, 128), lambda i: (0, i, 0)),
          pl.BlockSpec((msB.shape[0], BLK, DB), lambda i: (0, i, 0)),
          pl.BlockSpec((BLK, 1), lambda i: (i, 0)),
          full(Wg1), full(Wg2), full(Wg3), full(bg_a[None]),
          full(We1), full(We2), full(We3), full(be_a[None]),
          full(WgS), full(WgD), full(bgS),
          full(WeP), full(WeQ), full(be_e[None]),
      ],
      out_specs=[
          pl.BlockSpec((BLK, 128), lambda i: (i, 0)),
          pl.BlockSpec((NEXP + 1, BLK, 128), lambda i: (0, i, 0)),
          pl.BlockSpec((NEXP + 1, BLK, 128), lambda i: (0, i, 0)),
      ],
      out_shape=[
          jax.ShapeDtypeStruct((N, 128), jnp.float32),
          jax.ShapeDtypeStruct((NEXP + 1, N, 128), jnp.float32),
          jax.ShapeDtypeStruct((NEXP + 1, N, 128), jnp.float32),
      ],
  )(nf, msA, msB, deg, Wg1, Wg2, Wg3, bg_a[None], We1, We2, We3, be_a[None],
    WgS, WgD, bgS, WeP, WeQ, be_e[None])
  return outs


def _seg_gather_scatter_jnp(table, src, dst, n):
  s = jax.ops.segment_sum(table[src], dst, num_segments=n)
  return s[None]


def _seg_scatter_jnp(rows, dst, n):
  s = jax.ops.segment_sum(rows, dst, num_segments=n)
  return s[None]


def _seg_gather_scatter(table, src, dst, n):
  E = src.shape[0]
  NC, NS = _sc_info()
  HALF = n // NC
  ACC = HALF + 16
  ET = E // NS
  C2 = 80
  ZR = (ACC // NS) // 8 * 8
  ZTAIL = ACC - NS * ZR
  OR_ = (HALF // NS) // 8 * 8
  OTAIL = HALF - NS * OR_
  zeros = jnp.zeros((max(ZR, ZTAIL), 128), jnp.float32)

  @functools.partial(
      pl.kernel,
      out_type=jax.ShapeDtypeStruct((n, 128), jnp.float32),
      mesh=_sc_mesh(),
      scratch_types=[
          pltpu.VMEM((C2,), jnp.int32),
          pltpu.VMEM((C2,), jnp.int32),
          pltpu.VMEM((C2,), jnp.int32),
          pltpu.VMEM((C2, 128), jnp.float32),
          pltpu.VMEM_SHARED((ACC, 128), jnp.float32),
          pltpu.SemaphoreType.DMA,
      ],
  )
  def seg(table_h, src_h, dst_h, z_h, out_h, src_v, dst_v, dloc_v, rows_v,
          acc_sh, sem):
    c = lax.axis_index("c")
    sid = lax.axis_index("s")
    lo = c * HALF
    pltpu.sync_copy(z_h.at[pl.ds(0, ZR)], acc_sh.at[pl.ds(sid * ZR, ZR)])

    @pl.when(sid == NS - 1)
    def _zero_tail():
      pltpu.sync_copy(z_h.at[pl.ds(0, ZTAIL)],
                      acc_sh.at[pl.ds(NS * ZR, ZTAIL)])

    plsc.subcore_barrier()

    def chunk(ci, carry):
      base = sid * ET + ci * C2
      pltpu.sync_copy(src_h.at[pl.ds(base, C2)], src_v)
      pltpu.sync_copy(dst_h.at[pl.ds(base, C2)], dst_v)
      for g in range(C2 // LANES):
        sl = pl.ds(g * LANES, LANES)
        loc = dst_v[sl] - lo
        ok = (loc >= 0) & (loc < HALF)
        dloc_v[sl] = jnp.where(ok, loc, HALF)
      pltpu.async_copy(table_h.at[src_v], rows_v, sem).wait()
      pltpu.sync_copy(rows_v, acc_sh.at[dloc_v], add=True)
      return carry

    lax.fori_loop(0, ET // C2, chunk, 0)
    plsc.subcore_barrier()
    pltpu.sync_copy(acc_sh.at[pl.ds(sid * OR_, OR_)],
                    out_h.at[pl.ds(lo + sid * OR_, OR_)])

    @pl.when(sid == NS - 1)
    def _out_tail():
      pltpu.sync_copy(acc_sh.at[pl.ds(NS * OR_, OTAIL)],
                      out_h.at[pl.ds(lo + NS * OR_, OTAIL)])

  return seg(table, src, dst, zeros)[None]


def _seg_scatter(rows, dst, n):
  E, D = rows.shape
  NC, NS = _sc_info()
  HALF = n // NC
  ACC = HALF + 16
  ET = E // NS
  C2 = 80
  ZR = (ACC // NS) // 8 * 8
  ZTAIL = ACC - NS * ZR
  OR_ = (HALF // NS) // 8 * 8
  OTAIL = HALF - NS * OR_
  zeros = jnp.zeros((max(ZR, ZTAIL, C2), 128), jnp.float32)

  @functools.partial(
      pl.kernel,
      out_type=jax.ShapeDtypeStruct((n, 128), jnp.float32),
      mesh=_sc_mesh(),
      scratch_types=[
          pltpu.VMEM((C2,), jnp.int32),
          pltpu.VMEM((C2,), jnp.int32),
          pltpu.VMEM((C2, D), jnp.float32),
          pltpu.VMEM((C2, 128), jnp.float32),
          pltpu.VMEM_SHARED((ACC, 128), jnp.float32),
          pltpu.SemaphoreType.DMA,
      ],
  )
  def seg(rows_h, dst_h, z_h, out_h, dst_v, dloc_v, rows_s, rows_v, acc_sh,
          sem):
    c = lax.axis_index("c")
    sid = lax.axis_index("s")
    lo = c * HALF
    pltpu.sync_copy(z_h.at[pl.ds(0, ZR)], acc_sh.at[pl.ds(sid * ZR, ZR)])

    @pl.when(sid == NS - 1)
    def _zero_tail():
      pltpu.sync_copy(z_h.at[pl.ds(0, ZTAIL)],
                      acc_sh.at[pl.ds(NS * ZR, ZTAIL)])

    if D < 128:
      pltpu.sync_copy(z_h.at[pl.ds(0, C2)], rows_v)
    plsc.subcore_barrier()

    def chunk(ci, carry):
      base = sid * ET + ci * C2
      pltpu.sync_copy(dst_h.at[pl.ds(base, C2)], dst_v)
      for g in range(C2 // LANES):
        sl = pl.ds(g * LANES, LANES)
        loc = dst_v[sl] - lo
        ok = (loc >= 0) & (loc < HALF)
        dloc_v[sl] = jnp.where(ok, loc, HALF)
      if D < 128:
        pltpu.sync_copy(rows_h.at[pl.ds(base, C2)], rows_s)

        def expand(e, c2):
          for k in range(D // LANES):
            rows_v[e, pl.ds(k * LANES, LANES)] = rows_s[e,
                                                        pl.ds(k * LANES,
                                                              LANES)]
          return c2

        lax.fori_loop(0, C2, expand, 0)
      else:
        pltpu.sync_copy(rows_h.at[pl.ds(base, C2)], rows_v)
      pltpu.sync_copy(rows_v, acc_sh.at[dloc_v], add=True)
      return carry

    lax.fori_loop(0, ET // C2, chunk, 0)
    plsc.subcore_barrier()
    pltpu.sync_copy(acc_sh.at[pl.ds(sid * OR_, OR_)],
                    out_h.at[pl.ds(lo + sid * OR_, OR_)])

    @pl.when(sid == NS - 1)
    def _out_tail():
      pltpu.sync_copy(acc_sh.at[pl.ds(NS * OR_, OTAIL)],
                      out_h.at[pl.ds(lo + NS * OR_, OTAIL)])

  return seg(rows, dst, zeros)[None]


def _edge_combine(src, dst, P, Q, n):
  E = src.shape[0]
  NC, NS = _sc_info()
  NW = NC * NS
  EW = E // NW
  C = 80
  NCH = EW // C
  Pf = P.reshape((NEXP + 1) * n, HID)
  Qf = Q.reshape((NEXP + 1) * n, HID)
  buf = lambda shape, dt=jnp.float32: pltpu.VMEM(shape, dt)

  @functools.partial(
      pl.kernel,
      out_type=jax.ShapeDtypeStruct((E * HID,), jnp.float32),
      mesh=_sc_mesh(),
      scratch_types=[
          buf((C,), jnp.int32), buf((C,), jnp.int32),
          buf((C,), jnp.int32), buf((C,), jnp.int32),
          buf((C,), jnp.int32), buf((C,), jnp.int32),
          buf((C,), jnp.int32), buf((C,), jnp.int32),
          buf((C, HID)), buf((C, HID)),
          buf((C, HID)), buf((C, HID)),
          buf((C,)), buf((C,)),
          buf((C,), jnp.int32), buf((C,), jnp.int32),
          buf((C,), jnp.int32), buf((C,), jnp.int32),
          buf((C, HID)), buf((C, HID)),
          buf((C, HID)), buf((C, HID)),
          buf((C * HID,)),
          pltpu.SemaphoreType.DMA, pltpu.SemaphoreType.DMA,
          pltpu.SemaphoreType.DMA, pltpu.SemaphoreType.DMA,
      ],
  )
  def combine(src_h, dst_h, pf_h, qf_h, out_h,
              srcA, dstA, srcB, dstB, fA1, fA2, fB1, fB2,
              gsA, gdA, gsB, gdB, g1_v, g2_v,
              fi1, fi2, fi3, fi4, b1_v, b2_v, b3_v, b4_v, out_v,
              semI, semG, semP, semO):
    wid = lax.axis_index("s") * NC + lax.axis_index("c")
    ninf = jnp.full((LANES,), -jnp.inf, jnp.float32)
    zero_i = jnp.zeros((LANES,), jnp.int32)
    iota = lax.iota(jnp.int32, LANES)
    perms = {s: iota ^ s for s in (1, 2, 4, 8)}
    masks = {s: (iota & s) != 0 for s in (1, 2, 4, 8)}

    def _take(v, idx):
      return lax.gather(
          v, idx[:, None],
          lax.GatherDimensionNumbers(offset_dims=(), collapsed_slice_dims=(0,),
                                     start_index_map=(0,)),
          slice_sizes=(1,),
          mode=lax.GatherScatterMode.PROMISE_IN_BOUNDS)

    def idx_slice(ci):
      return pl.ds(wid * EW + ci * C, C)

    def gate_idx(sv, dv, f1, f2):
      for g in range(C // LANES):
        sl = pl.ds(g * LANES, LANES)
        f1[sl] = sv[sl] + NEXP * n
        f2[sl] = dv[sl] + NEXP * n

    def fire_G(f1, f2, gs, gd):
      pltpu.async_copy(pf_h.at[f1], gs, semG)
      pltpu.async_copy(qf_h.at[f2], gd, semG)

    def drain_G(f1, f2, gs, gd):
      pltpu.make_async_copy(pf_h.at[f1], gs, semG).wait()
      pltpu.make_async_copy(qf_h.at[f2], gd, semG).wait()

    def process(ci, cur, nxt, has_next, first, last):
      sv, dv, f1, f2, gs, gd = cur
      if has_next:
        pltpu.async_copy(src_h.at[idx_slice(ci + 1)], nxt[0], semI)
        pltpu.async_copy(dst_h.at[idx_slice(ci + 1)], nxt[1], semI)
      drain_G(f1, f2, gs, gd)
      for g in range(C // LANES):
        R = [gs[g * LANES + l, pl.ds(0, LANES)]
             + gd[g * LANES + l, pl.ds(0, LANES)]
             for l in range(LANES)]
        for s in (8, 4, 2, 1):
          pm, mk = perms[s], masks[s]
          for i in range(LANES):
            if i & s:
              continue
            a, b = R[i], R[i | s]
            R[i] = jnp.where(mk, _take(b, pm), a)
            R[i | s] = jnp.where(mk, b, _take(a, pm))
        m1 = R[0]
        i1 = zero_i
        m2 = ninf
        i2 = zero_i
        for j in range(1, NEXP):
          x = R[j]
          cj = zero_i + j
          gt1 = x > m1
          gt2 = x > m2
          i2 = jnp.where(gt1, i1, jnp.where(gt2, cj, i2))
          m2 = jnp.where(gt1, m1, jnp.where(gt2, x, m2))
          i1 = jnp.where(gt1, cj, i1)
          m1 = jnp.where(gt1, x, m1)
        e2 = jnp.exp(m2 - m1)
        g1 = 1.0 / (1.0 + e2)
        sl = pl.ds(g * LANES, LANES)
        s16 = sv[sl]
        d16 = dv[sl]
        g1_v[sl] = g1
        g2_v[sl] = 1.0 - g1
        fi1[sl] = i1 * n + s16
        fi2[sl] = i1 * n + d16
        fi3[sl] = i2 * n + s16
        fi4[sl] = i2 * n + d16
      d1 = pltpu.async_copy(pf_h.at[fi1], b1_v, semP)
      d2 = pltpu.async_copy(qf_h.at[fi2], b2_v, semP)
      d3 = pltpu.async_copy(pf_h.at[fi3], b3_v, semP)
      d4 = pltpu.async_copy(qf_h.at[fi4], b4_v, semP)
      if has_next:
        pltpu.make_async_copy(src_h.at[idx_slice(ci + 1)], nxt[0], semI).wait()
        pltpu.make_async_copy(dst_h.at[idx_slice(ci + 1)], nxt[1], semI).wait()
        gate_idx(nxt[0], nxt[1], nxt[2], nxt[3])
        fire_G(nxt[2], nxt[3], nxt[4], nxt[5])
      d1.wait()
      d2.wait()
      d3.wait()
      d4.wait()
      pltpu.make_async_copy(out_v, out_h.at[pl.ds(0, C * HID)], semO).wait()

      def edge_body(e, c2):
        b16 = (e // LANES) * LANES
        off = e - b16
        offv = zero_i + off
        g1b = _take(g1_v[pl.ds(b16, LANES)], offv)
        g2b = _take(g2_v[pl.ds(b16, LANES)], offv)
        for k in range(HID // LANES):
          sk = pl.ds(k * LANES, LANES)
          r = (g1b * (b1_v[e, sk] + b2_v[e, sk])
               + g2b * (b3_v[e, sk] + b4_v[e, sk]))
          out_v[pl.ds(e * HID + k * LANES, LANES)] = jnp.maximum(r, 0.0)
        return c2

      lax.fori_loop(0, C, edge_body, 0)
      obase = (wid * EW + ci * C) * HID
      if last:
        pltpu.sync_copy(out_v, out_h.at[pl.ds(obase, C * HID)])
      else:
        pltpu.async_copy(out_v, out_h.at[pl.ds(obase, C * HID)], semO)

    A = (srcA, dstA, fA1, fA2, gsA, gdA)
    B = (srcB, dstB, fB1, fB2, gsB, gdB)
    pltpu.sync_copy(src_h.at[idx_slice(0)], srcA)
    pltpu.sync_copy(dst_h.at[idx_slice(0)], dstA)
    gate_idx(srcA, dstA, fA1, fA2)
    fire_G(fA1, fA2, gsA, gdA)
    pltpu.async_copy(out_v, out_h.at[pl.ds(wid * EW * HID, C * HID)], semO)

    def pair_body(j, carry):
      process(2 * j, A, B, True, False, False)
      process(2 * j + 1, B, A, True, False, False)
      return carry

    lax.fori_loop(0, (NCH - 1) // 2, pair_body, 0)
    process(NCH - 1, A, B, False, False, True)

  return combine(src, dst, Pf, Qf).reshape(E, HID)


def kernel(edge_index, nfeats, efeats,
           Wg_a0, bg_a0, We_a0, be_a0, Wg_e0, bg_e0, We_e0, be_e0,
           Wg_a1, bg_a1, We_a1, be_a1, Wg_e1, bg_e1, We_e1, be_e1):
  n = nfeats.shape[0]
  e = efeats.shape[0]
  src = edge_index[0].astype(jnp.int32)
  dst = edge_index[1].astype(jnp.int32)

  msA0 = _seg_gather_scatter(nfeats, src, dst, n)
  ef_aug = jnp.concatenate(
      [efeats, jnp.ones((e, 1), jnp.float32),
       jnp.zeros((e, 15), jnp.float32)], axis=1)
  msB0 = _seg_scatter(ef_aug, dst, n)
  deg = msB0.sum(axis=0)[:, 16:17]

  hn0, P0, Q0 = _node_moe_prec(
      nfeats, msA0, msB0, deg, Wg_a0, bg_a0, We_a0, be_a0,
      Wg_e0, bg_e0, We_e0, be_e0, DB_use=16)
  he0 = _edge_combine(src, dst, P0, Q0, n)

  msA1 = _seg_gather_scatter(hn0, src, dst, n)
  msB1 = _seg_scatter(he0, dst, n)
  hn1, P1, Q1 = _node_moe_prec(
      hn0, msA1, msB1, deg, Wg_a1, bg_a1, We_a1, be_a1,
      Wg_e1, bg_e1, We_e1, be_e1, DB_use=128)
  he1 = _edge_combine(src, dst, P1, Q1, n)
  return (hn1, he1)


# speedup vs baseline: 62.2208x; 1.2590x over previous
import functools

import jax
import jax.numpy as jnp
from jax import lax
from jax.experimental import pallas as pl
from jax.experimental.pallas import tpu as pltpu
from jax.experimental.pallas import tpu_sc as plsc

NEXP = 8
HID = 128
LANES = 16


def _sc_mesh():
  return plsc.VectorSubcoreMesh(core_axis_name="c", subcore_axis_name="s")


def _sc_info():
  try:
    info = plsc.get_sparse_core_info()
    return info.num_cores, info.num_subcores
  except Exception:
    return 2, 16


def _node_moe_body(DB_use,
                   nf_ref, msA_ref, msB_ref, deg_ref,
                   Wg1, Wg2, Wg3, bg,
                   We1, We2, We3, be,
                   WgS, WgD, bgS,
                   WeP, WeQ, beE,
                   hn_out, P_out, Q_out):
  nf = nf_ref[...]
  invd = 1.0 / jnp.maximum(deg_ref[...], 1.0)
  hA = sum(msA_ref[k] for k in range(msA_ref.shape[0])) * invd
  hB = sum(msB_ref[k, :, :DB_use] for k in range(msB_ref.shape[0])) * invd
  logits = nf @ Wg1[...] + hA @ Wg2[...] + hB @ Wg3[...] + bg[...]
  blk = logits.shape[0]
  ids = lax.broadcasted_iota(jnp.int32, (blk, NEXP), 1)
  v1 = jnp.max(logits, axis=-1, keepdims=True)
  i1 = jnp.argmax(logits, axis=-1).astype(jnp.int32)[:, None]
  l2 = jnp.where(ids == i1, -jnp.inf, logits)
  v2 = jnp.max(l2, axis=-1, keepdims=True)
  i2 = jnp.argmax(l2, axis=-1).astype(jnp.int32)[:, None]
  e2 = jnp.exp(v2 - v1)
  den = 1.0 + e2
  g1 = 1.0 / den
  g2 = e2 / den
  acc = jnp.zeros((blk, HID), jnp.float32)
  for e in range(NEXP):
    xe = nf @ We1[e] + hA @ We2[e] + hB @ We3[e] + be[0, e]
    w = jnp.where(i1 == e, g1, 0.0) + jnp.where(i2 == e, g2, 0.0)
    acc = acc + w * xe
  hn = jnp.maximum(acc, 0.0)
  hn_out[...] = hn
  for e in range(NEXP):
    P_out[e] = hn @ WeP[e] + 0.5 * beE[0, e]
    Q_out[e] = hn @ WeQ[e] + 0.5 * beE[0, e]
  P_out[NEXP] = hn @ WgS[...] + bgS[...]
  Q_out[NEXP] = hn @ WgD[...]


def _node_moe_prec(nf, msA, msB, deg, Wg_a, bg_a, We_a, be_a,
                   Wg_e, bg_e, We_e, be_e, DB_use):
  N = nf.shape[0]
  DB = msB.shape[2]
  DM = Wg_a.shape[0]
  BLK = 400 if N % 400 == 0 else N
  grid = (N // BLK,)
  Wg1 = Wg_a[:128]
  Wg2 = Wg_a[128:256]
  Wg3 = Wg_a[256:DM]
  We1 = We_a[:, :128, :]
  We2 = We_a[:, 128:256, :]
  We3 = We_a[:, 256:DM, :]
  WgS = jnp.pad(Wg_e[:128], ((0, 0), (0, 120)))
  WgD = jnp.pad(Wg_e[128:], ((0, 0), (0, 120)))
  bgS = jnp.pad(bg_e, (0, 120))[None]
  WeP = We_e[:, :128, :]
  WeQ = We_e[:, 128:, :]

  full = lambda a: pl.BlockSpec(a.shape, lambda i: (0,) * a.ndim)
  outs = pl.pallas_call(
      functools.partial(_node_moe_body, DB_use),
      grid=grid,
      in_specs=[
          pl.BlockSpec((BLK, 128), lambda i: (i, 0)),
          pl.BlockSpec((msA.shape[0], BLK, 128), lambda i: (0, i, 0)),
          pl.BlockSpec((msB.shape[0], BLK, DB), lambda i: (0, i, 0)),
          pl.BlockSpec((BLK, 1), lambda i: (i, 0)),
          full(Wg1), full(Wg2), full(Wg3), full(bg_a[None]),
          full(We1), full(We2), full(We3), full(be_a[None]),
          full(WgS), full(WgD), full(bgS),
          full(WeP), full(WeQ), full(be_e[None]),
      ],
      out_specs=[
          pl.BlockSpec((BLK, 128), lambda i: (i, 0)),
          pl.BlockSpec((NEXP + 1, BLK, 128), lambda i: (0, i, 0)),
          pl.BlockSpec((NEXP + 1, BLK, 128), lambda i: (0, i, 0)),
      ],
      out_shape=[
          jax.ShapeDtypeStruct((N, 128), jnp.float32),
          jax.ShapeDtypeStruct((NEXP + 1, N, 128), jnp.float32),
          jax.ShapeDtypeStruct((NEXP + 1, N, 128), jnp.float32),
      ],
  )(nf, msA, msB, deg, Wg1, Wg2, Wg3, bg_a[None], We1, We2, We3, be_a[None],
    WgS, WgD, bgS, WeP, WeQ, be_e[None])
  return outs


def _seg_gather_scatter_jnp(table, src, dst, n):
  s = jax.ops.segment_sum(table[src], dst, num_segments=n)
  return s[None]


def _seg_scatter_jnp(rows, dst, n):
  s = jax.ops.segment_sum(rows, dst, num_segments=n)
  return s[None]


def _seg_gather_scatter(table, src, dst, n):
  E = src.shape[0]
  NC, NS = _sc_info()
  HALF = n // NC
  ACC = HALF + 16
  ET = E // NS
  C2 = 80
  NCH = ET // C2
  ZR = (ACC // NS) // 8 * 8
  ZTAIL = ACC - NS * ZR
  OR_ = (HALF // NS) // 8 * 8
  OTAIL = HALF - NS * OR_
  zeros = jnp.zeros((max(ZR, ZTAIL), 128), jnp.float32)
  buf = lambda shape, dt=jnp.float32: pltpu.VMEM(shape, dt)

  @functools.partial(
      pl.kernel,
      out_type=jax.ShapeDtypeStruct((n, 128), jnp.float32),
      mesh=_sc_mesh(),
      scratch_types=[
          buf((C2,), jnp.int32), buf((C2,), jnp.int32),
          buf((C2,), jnp.int32), buf((C2,), jnp.int32),
          buf((C2,), jnp.int32), buf((C2,), jnp.int32),
          buf((C2, 128)), buf((C2, 128)),
          pltpu.VMEM_SHARED((ACC, 128), jnp.float32),
          pltpu.SemaphoreType.DMA, pltpu.SemaphoreType.DMA,
          pltpu.SemaphoreType.DMA,
      ],
  )
  def seg(table_h, src_h, dst_h, z_h, out_h,
          srcA, dstA, srcB, dstB, dlocA, dlocB, rowsA, rowsB, acc_sh,
          semI, semT, semS):
    c = lax.axis_index("c")
    sid = lax.axis_index("s")
    lo = c * HALF
    pltpu.sync_copy(z_h.at[pl.ds(0, ZR)], acc_sh.at[pl.ds(sid * ZR, ZR)])

    @pl.when(sid == NS - 1)
    def _zero_tail():
      pltpu.sync_copy(z_h.at[pl.ds(0, ZTAIL)],
                      acc_sh.at[pl.ds(NS * ZR, ZTAIL)])

    plsc.subcore_barrier()

    def idx_slice(ci):
      return pl.ds(sid * ET + ci * C2, C2)

    def process(ci, cur, nxt, has_next, first):
      sv, dv, dl, rv = cur
      if has_next:
        pltpu.async_copy(src_h.at[idx_slice(ci + 1)], nxt[0], semI)
        pltpu.async_copy(dst_h.at[idx_slice(ci + 1)], nxt[1], semI)
      pltpu.make_async_copy(table_h.at[pl.ds(0, C2)], rv, semT).wait()
      for g in range(C2 // LANES):
        sl = pl.ds(g * LANES, LANES)
        loc = dv[sl] - lo
        ok = (loc >= 0) & (loc < HALF)
        dl[sl] = jnp.where(ok, loc, HALF)
      pltpu.async_copy(rv, acc_sh.at[dl], semS, add=True)
      if has_next:
        pltpu.make_async_copy(src_h.at[idx_slice(ci + 1)], nxt[0], semI).wait()
        pltpu.make_async_copy(dst_h.at[idx_slice(ci + 1)], nxt[1], semI).wait()
        if not first:
          pltpu.make_async_copy(nxt[3], acc_sh.at[pl.ds(0, C2)], semS).wait()
        pltpu.async_copy(table_h.at[nxt[0]], nxt[3], semT)

    A = (srcA, dstA, dlocA, rowsA)
    B = (srcB, dstB, dlocB, rowsB)
    pltpu.sync_copy(src_h.at[idx_slice(0)], srcA)
    pltpu.sync_copy(dst_h.at[idx_slice(0)], dstA)
    pltpu.async_copy(table_h.at[srcA], rowsA, semT)
    process(0, A, B, True, True)
    process(1, B, A, True, False)

    def pair_body(j, carry):
      process(2 * j, A, B, True, False)
      process(2 * j + 1, B, A, True, False)
      return carry

    lax.fori_loop(1, NCH // 2 - 1, pair_body, 0)
    process(NCH - 2, A, B, True, False)
    process(NCH - 1, B, A, False, False)
    pltpu.make_async_copy(rowsA, acc_sh.at[pl.ds(0, C2)], semS).wait()
    pltpu.make_async_copy(rowsB, acc_sh.at[pl.ds(0, C2)], semS).wait()
    plsc.subcore_barrier()
    pltpu.sync_copy(acc_sh.at[pl.ds(sid * OR_, OR_)],
                    out_h.at[pl.ds(lo + sid * OR_, OR_)])

    @pl.when(sid == NS - 1)
    def _out_tail():
      pltpu.sync_copy(acc_sh.at[pl.ds(NS * OR_, OTAIL)],
                      out_h.at[pl.ds(lo + NS * OR_, OTAIL)])

  return seg(table, src, dst, zeros)[None]


def _seg_scatter(rows, dst, n):
  E, D = rows.shape
  NC, NS = _sc_info()
  HALF = n // NC
  ACC = HALF + 16
  ET = E // NS
  C2 = 80
  NCH = ET // C2
  ZR = (ACC // NS) // 8 * 8
  ZTAIL = ACC - NS * ZR
  OR_ = (HALF // NS) // 8 * 8
  OTAIL = HALF - NS * OR_
  zeros = jnp.zeros((max(ZR, ZTAIL, C2), 128), jnp.float32)
  narrow = D < 128
  buf = lambda shape, dt=jnp.float32: pltpu.VMEM(shape, dt)
  stage_shape = (C2, D) if narrow else (C2, 128)

  @functools.partial(
      pl.kernel,
      out_type=jax.ShapeDtypeStruct((n, 128), jnp.float32),
      mesh=_sc_mesh(),
      scratch_types=[
          buf((C2,), jnp.int32), buf((C2,), jnp.int32),
          buf((C2,), jnp.int32), buf((C2,), jnp.int32),
          buf(stage_shape), buf(stage_shape),
          buf((C2, 128)), buf((C2, 128)),
          pltpu.VMEM_SHARED((ACC, 128), jnp.float32),
          pltpu.SemaphoreType.DMA, pltpu.SemaphoreType.DMA,
          pltpu.SemaphoreType.DMA,
      ],
  )
  def seg(rows_h, dst_h, z_h, out_h,
          dstA, dstB, dlocA, dlocB, stA, stB, wdA, wdB, acc_sh,
          semI, semR, semS):
    c = lax.axis_index("c")
    sid = lax.axis_index("s")
    lo = c * HALF
    pltpu.sync_copy(z_h.at[pl.ds(0, ZR)], acc_sh.at[pl.ds(sid * ZR, ZR)])

    @pl.when(sid == NS - 1)
    def _zero_tail():
      pltpu.sync_copy(z_h.at[pl.ds(0, ZTAIL)],
                      acc_sh.at[pl.ds(NS * ZR, ZTAIL)])

    if narrow:
      pltpu.sync_copy(z_h.at[pl.ds(0, C2)], wdA)
      pltpu.sync_copy(z_h.at[pl.ds(0, C2)], wdB)
    plsc.subcore_barrier()

    def idx_slice(ci):
      return pl.ds(sid * ET + ci * C2, C2)

    def process(ci, cur, nxt, has_next, first):
      dv, dl, st, wd = cur
      if has_next:
        pltpu.async_copy(dst_h.at[idx_slice(ci + 1)], nxt[0], semI)
      pltpu.make_async_copy(rows_h.at[idx_slice(0)], st, semR).wait()
      for g in range(C2 // LANES):
        sl = pl.ds(g * LANES, LANES)
        loc = dv[sl] - lo
        ok = (loc >= 0) & (loc < HALF)
        dl[sl] = jnp.where(ok, loc, HALF)
      if narrow:
        def expand(e, c2):
          for k in range(D // LANES):
            wd[e, pl.ds(k * LANES, LANES)] = st[e, pl.ds(k * LANES, LANES)]
          return c2

        lax.fori_loop(0, C2, expand, 0)
      pltpu.async_copy(wd, acc_sh.at[dl], semS, add=True)
      if has_next:
        pltpu.make_async_copy(dst_h.at[idx_slice(ci + 1)], nxt[0],
                              semI).wait()
        if not first:
          pltpu.make_async_copy(nxt[3], acc_sh.at[pl.ds(0, C2)], semS).wait()
        pltpu.async_copy(rows_h.at[idx_slice(ci + 1)], nxt[2], semR)

    A = (dstA, dlocA, stA, wdA if narrow else stA)
    B = (dstB, dlocB, stB, wdB if narrow else stB)
    pltpu.sync_copy(dst_h.at[idx_slice(0)], dstA)
    pltpu.async_copy(rows_h.at[idx_slice(0)], stA, semR)
    process(0, A, B, True, True)
    process(1, B, A, True, False)

    def pair_body(j, carry):
      process(2 * j, A, B, True, False)
      process(2 * j + 1, B, A, True, False)
      return carry

    lax.fori_loop(1, NCH // 2 - 1, pair_body, 0)
    process(NCH - 2, A, B, True, False)
    process(NCH - 1, B, A, False, False)
    pltpu.make_async_copy(A[3], acc_sh.at[pl.ds(0, C2)], semS).wait()
    pltpu.make_async_copy(B[3], acc_sh.at[pl.ds(0, C2)], semS).wait()
    plsc.subcore_barrier()
    pltpu.sync_copy(acc_sh.at[pl.ds(sid * OR_, OR_)],
                    out_h.at[pl.ds(lo + sid * OR_, OR_)])

    @pl.when(sid == NS - 1)
    def _out_tail():
      pltpu.sync_copy(acc_sh.at[pl.ds(NS * OR_, OTAIL)],
                      out_h.at[pl.ds(lo + NS * OR_, OTAIL)])

  return seg(rows, dst, zeros)[None]


def _edge_combine(src, dst, P, Q, n):
  E = src.shape[0]
  NC, NS = _sc_info()
  NW = NC * NS
  EW = E // NW
  C = 80
  NCH = EW // C
  Pf = P.reshape((NEXP + 1) * n, HID)
  Qf = Q.reshape((NEXP + 1) * n, HID)
  buf = lambda shape, dt=jnp.float32: pltpu.VMEM(shape, dt)

  @functools.partial(
      pl.kernel,
      out_type=jax.ShapeDtypeStruct((E * HID,), jnp.float32),
      mesh=_sc_mesh(),
      scratch_types=[
          buf((C,), jnp.int32), buf((C,), jnp.int32),
          buf((C,), jnp.int32), buf((C,), jnp.int32),
          buf((C,), jnp.int32), buf((C,), jnp.int32),
          buf((C,), jnp.int32), buf((C,), jnp.int32),
          buf((C, HID)), buf((C, HID)),
          buf((C, HID)), buf((C, HID)),
          buf((C,)), buf((C,)),
          buf((C,), jnp.int32), buf((C,), jnp.int32),
          buf((C,), jnp.int32), buf((C,), jnp.int32),
          buf((C, HID)), buf((C, HID)),
          buf((C, HID)), buf((C, HID)),
          buf((C * HID,)),
          pltpu.SemaphoreType.DMA, pltpu.SemaphoreType.DMA,
          pltpu.SemaphoreType.DMA, pltpu.SemaphoreType.DMA,
      ],
  )
  def combine(src_h, dst_h, pf_h, qf_h, out_h,
              srcA, dstA, srcB, dstB, fA1, fA2, fB1, fB2,
              gsA, gdA, gsB, gdB, g1_v, g2_v,
              fi1, fi2, fi3, fi4, b1_v, b2_v, b3_v, b4_v, out_v,
              semI, semG, semP, semO):
    wid = lax.axis_index("s") * NC + lax.axis_index("c")
    ninf = jnp.full((LANES,), -jnp.inf, jnp.float32)
    zero_i = jnp.zeros((LANES,), jnp.int32)
    iota = lax.iota(jnp.int32, LANES)
    perms = {s: iota ^ s for s in (1, 2, 4, 8)}
    masks = {s: (iota & s) != 0 for s in (1, 2, 4, 8)}

    def _take(v, idx):
      return lax.gather(
          v, idx[:, None],
          lax.GatherDimensionNumbers(offset_dims=(), collapsed_slice_dims=(0,),
                                     start_index_map=(0,)),
          slice_sizes=(1,),
          mode=lax.GatherScatterMode.PROMISE_IN_BOUNDS)

    def idx_slice(ci):
      return pl.ds(wid * EW + ci * C, C)

    def gate_idx(sv, dv, f1, f2):
      for g in range(C // LANES):
        sl = pl.ds(g * LANES, LANES)
        f1[sl] = sv[sl] + NEXP * n
        f2[sl] = dv[sl] + NEXP * n

    def fire_G(f1, f2, gs, gd):
      pltpu.async_copy(pf_h.at[f1], gs, semG)
      pltpu.async_copy(qf_h.at[f2], gd, semG)

    def drain_G(f1, f2, gs, gd):
      pltpu.make_async_copy(pf_h.at[f1], gs, semG).wait()
      pltpu.make_async_copy(qf_h.at[f2], gd, semG).wait()

    def process(ci, cur, nxt, has_next, first, last):
      sv, dv, f1, f2, gs, gd = cur
      if has_next:
        pltpu.async_copy(src_h.at[idx_slice(ci + 1)], nxt[0], semI)
        pltpu.async_copy(dst_h.at[idx_slice(ci + 1)], nxt[1], semI)
      drain_G(f1, f2, gs, gd)
      for g in range(C // LANES):
        R = [gs[g * LANES + l, pl.ds(0, LANES)]
             + gd[g * LANES + l, pl.ds(0, LANES)]
             for l in range(LANES)]
        for s in (8, 4, 2, 1):
          pm, mk = perms[s], masks[s]
          for i in range(LANES):
            if i & s:
              continue
            a, b = R[i], R[i | s]
            R[i] = jnp.where(mk, _take(b, pm), a)
            R[i | s] = jnp.where(mk, b, _take(a, pm))
        m1 = R[0]
        i1 = zero_i
        m2 = ninf
        i2 = zero_i
        for j in range(1, NEXP):
          x = R[j]
          cj = zero_i + j
          gt1 = x > m1
          gt2 = x > m2
          i2 = jnp.where(gt1, i1, jnp.where(gt2, cj, i2))
          m2 = jnp.where(gt1, m1, jnp.where(gt2, x, m2))
          i1 = jnp.where(gt1, cj, i1)
          m1 = jnp.where(gt1, x, m1)
        e2 = jnp.exp(m2 - m1)
        g1 = 1.0 / (1.0 + e2)
        sl = pl.ds(g * LANES, LANES)
        s16 = sv[sl]
        d16 = dv[sl]
        g1_v[sl] = g1
        g2_v[sl] = 1.0 - g1
        fi1[sl] = i1 * n + s16
        fi2[sl] = i1 * n + d16
        fi3[sl] = i2 * n + s16
        fi4[sl] = i2 * n + d16
      d1 = pltpu.async_copy(pf_h.at[fi1], b1_v, semP)
      d2 = pltpu.async_copy(qf_h.at[fi2], b2_v, semP)
      d3 = pltpu.async_copy(pf_h.at[fi3], b3_v, semP)
      d4 = pltpu.async_copy(qf_h.at[fi4], b4_v, semP)
      if has_next:
        pltpu.make_async_copy(src_h.at[idx_slice(ci + 1)], nxt[0], semI).wait()
        pltpu.make_async_copy(dst_h.at[idx_slice(ci + 1)], nxt[1], semI).wait()
        gate_idx(nxt[0], nxt[1], nxt[2], nxt[3])
        fire_G(nxt[2], nxt[3], nxt[4], nxt[5])
      d1.wait()
      d2.wait()
      d3.wait()
      d4.wait()
      pltpu.make_async_copy(out_v, out_h.at[pl.ds(0, C * HID)], semO).wait()

      def edge_body(e, c2):
        b16 = (e // LANES) * LANES
        off = e - b16
        offv = zero_i + off
        g1b = _take(g1_v[pl.ds(b16, LANES)], offv)
        g2b = _take(g2_v[pl.ds(b16, LANES)], offv)
        for k in range(HID // LANES):
          sk = pl.ds(k * LANES, LANES)
          r = (g1b * (b1_v[e, sk] + b2_v[e, sk])
               + g2b * (b3_v[e, sk] + b4_v[e, sk]))
          out_v[pl.ds(e * HID + k * LANES, LANES)] = jnp.maximum(r, 0.0)
        return c2

      lax.fori_loop(0, C, edge_body, 0)
      obase = (wid * EW + ci * C) * HID
      if last:
        pltpu.sync_copy(out_v, out_h.at[pl.ds(obase, C * HID)])
      else:
        pltpu.async_copy(out_v, out_h.at[pl.ds(obase, C * HID)], semO)

    A = (srcA, dstA, fA1, fA2, gsA, gdA)
    B = (srcB, dstB, fB1, fB2, gsB, gdB)
    pltpu.sync_copy(src_h.at[idx_slice(0)], srcA)
    pltpu.sync_copy(dst_h.at[idx_slice(0)], dstA)
    gate_idx(srcA, dstA, fA1, fA2)
    fire_G(fA1, fA2, gsA, gdA)
    pltpu.async_copy(out_v, out_h.at[pl.ds(wid * EW * HID, C * HID)], semO)

    def pair_body(j, carry):
      process(2 * j, A, B, True, False, False)
      process(2 * j + 1, B, A, True, False, False)
      return carry

    lax.fori_loop(0, (NCH - 1) // 2, pair_body, 0)
    process(NCH - 1, A, B, False, False, True)

  return combine(src, dst, Pf, Qf).reshape(E, HID)


def kernel(edge_index, nfeats, efeats,
           Wg_a0, bg_a0, We_a0, be_a0, Wg_e0, bg_e0, We_e0, be_e0,
           Wg_a1, bg_a1, We_a1, be_a1, Wg_e1, bg_e1, We_e1, be_e1):
  n = nfeats.shape[0]
  e = efeats.shape[0]
  src = edge_index[0].astype(jnp.int32)
  dst = edge_index[1].astype(jnp.int32)

  msA0 = _seg_gather_scatter(nfeats, src, dst, n)
  ef_aug = jnp.concatenate(
      [efeats, jnp.ones((e, 1), jnp.float32),
       jnp.zeros((e, 15), jnp.float32)], axis=1)
  msB0 = _seg_scatter(ef_aug, dst, n)
  deg = msB0.sum(axis=0)[:, 16:17]

  hn0, P0, Q0 = _node_moe_prec(
      nfeats, msA0, msB0, deg, Wg_a0, bg_a0, We_a0, be_a0,
      Wg_e0, bg_e0, We_e0, be_e0, DB_use=16)
  he0 = _edge_combine(src, dst, P0, Q0, n)

  msA1 = _seg_gather_scatter(hn0, src, dst, n)
  msB1 = _seg_scatter(he0, dst, n)
  hn1, P1, Q1 = _node_moe_prec(
      hn0, msA1, msB1, deg, Wg_a1, bg_a1, We_a1, be_a1,
      Wg_e1, bg_e1, We_e1, be_e1, DB_use=128)
  he1 = _edge_combine(src, dst, P1, Q1, n)
  return (hn1, he1)
